# baseline (speedup 1.0000x reference)
import functools

import jax
import jax.numpy as jnp
from jax import lax
from jax.experimental import pallas as pl
from jax.experimental.pallas import tpu as pltpu
from jax.experimental.pallas import tpu_sc as plsc

_VOCAB = 1000000
_EMBED = 64
_B = 16384
_L = 20
_NC = 2
_NS = 16
_NW = _NC * _NS
_ROWS_W = _B // _NW
_RBLK = 16
_NBLK = _ROWS_W // _RBLK

_mesh = plsc.VectorSubcoreMesh(core_axis_name="c", subcore_axis_name="s")


@functools.partial(
    pl.kernel,
    mesh=_mesh,
    out_type=jax.ShapeDtypeStruct((_B, _L, _EMBED), jnp.float32),
    scratch_types=[
        pltpu.VMEM((_RBLK, _L), jnp.int32),
        pltpu.VMEM((_RBLK, _L, _EMBED), jnp.float32),
        pltpu.SemaphoreType.DMA,
    ],
    compiler_params=pltpu.CompilerParams(use_tc_tiling_on_sc=False),
)
def _embed_sc(x_hbm, table_hbm, out_hbm, idx_v, rows_v, sem):
    wid = lax.axis_index("s") * _NC + lax.axis_index("c")
    row0 = wid * _ROWS_W

    def body(i, carry):
        r = row0 + i * _RBLK
        pltpu.sync_copy(x_hbm.at[pl.ds(r, _RBLK), :], idx_v)
        descs = [
            pltpu.async_copy(table_hbm.at[idx_v.at[j]], rows_v.at[j], sem)
            for j in range(_RBLK)
        ]
        for d in descs:
            d.wait()
        pltpu.sync_copy(rows_v, out_hbm.at[pl.ds(r, _RBLK), :, :])
        return carry

    lax.fori_loop(0, _NBLK, body, 0)


def kernel(x, W):
    return _embed_sc(x.astype(jnp.int32), W)


# speedup vs baseline: 1.0255x; 1.0255x over previous
import functools

import jax
import jax.numpy as jnp
from jax import lax
from jax.experimental import pallas as pl
from jax.experimental.pallas import tpu as pltpu
from jax.experimental.pallas import tpu_sc as plsc

_VOCAB = 1000000
_EMBED = 64
_B = 16384
_L = 20
_N = _B * _L
_NC = 2
_NS = 16
_NW = _NC * _NS
_ROWS_W = _B // _NW
_PER_W = _N // _NW
_RBLK = 32
_CH = _RBLK * _L
_NCHUNK = _ROWS_W // _RBLK

_mesh = plsc.VectorSubcoreMesh(core_axis_name="c", subcore_axis_name="s")


@functools.partial(
    pl.kernel,
    mesh=_mesh,
    out_type=jax.ShapeDtypeStruct((_N,), jnp.int32),
    scratch_types=[
        pltpu.VMEM((_ROWS_W, _L), jnp.int32),
        pltpu.VMEM((_PER_W,), jnp.int32),
    ],
    compiler_params=pltpu.CompilerParams(needs_layout_passes=False),
)
def _flatten_sc(x_hbm, xf_hbm, xv, fv):
    wid = lax.axis_index("s") * _NC + lax.axis_index("c")
    r0 = wid * _ROWS_W
    pltpu.sync_copy(x_hbm.at[pl.ds(r0, _ROWS_W), :], xv)

    def body(r, carry):
        i16 = lax.iota(jnp.int32, 16)
        base = r * _L
        v1 = xv[r, pl.ds(0, 16)]
        plsc.store_scatter(fv, [base + i16], v1)
        v2 = xv[r, pl.ds(4, 16)]
        plsc.store_scatter(fv, [base + 4 + i16], v2, mask=i16 >= 12)
        return carry

    lax.fori_loop(0, _ROWS_W, body, 0)
    pltpu.sync_copy(fv, xf_hbm.at[pl.ds(wid * _PER_W, _PER_W)])


@functools.partial(
    pl.kernel,
    mesh=_mesh,
    out_type=jax.ShapeDtypeStruct((_B, _L, _EMBED), jnp.float32),
    scratch_types=[
        pltpu.VMEM((_CH,), jnp.int32),
        pltpu.VMEM((_CH, _EMBED), jnp.float32),
        pltpu.SemaphoreType.DMA,
        pltpu.SemaphoreType.DMA,
    ],
    compiler_params=pltpu.CompilerParams(use_tc_tiling_on_sc=False),
)
def _embed_sc(xf_hbm, table_hbm, out_hbm, idx_v, rows_v, gsem, ssem):
    wid = lax.axis_index("s") * _NC + lax.axis_index("c")
    base = wid * _PER_W
    row0 = wid * _ROWS_W

    def body(i, carry):
        pltpu.sync_copy(xf_hbm.at[pl.ds(base + i * _CH, _CH)], idx_v)
        pltpu.async_copy(table_hbm.at[idx_v], rows_v, gsem).wait()
        r = row0 + i * _RBLK
        descs = [
            pltpu.async_copy(
                rows_v.at[pl.ds(k * _L, _L), :], out_hbm.at[r + k], ssem
            )
            for k in range(_RBLK)
        ]
        for d in descs:
            d.wait()
        return carry

    lax.fori_loop(0, _NCHUNK, body, 0)


def kernel(x, W):
    xf = _flatten_sc(x.astype(jnp.int32))
    return _embed_sc(xf, W)


# speedup vs baseline: 1.0267x; 1.0012x over previous
import functools

import jax
import jax.numpy as jnp
from jax import lax
from jax.experimental import pallas as pl
from jax.experimental.pallas import tpu as pltpu
from jax.experimental.pallas import tpu_sc as plsc

_VOCAB = 1000000
_EMBED = 64
_B = 16384
_L = 20
_N = _B * _L
_NC = 2
_NS = 16
_NW = _NC * _NS
_ROWS_W = _B // _NW
_PER_W = _N // _NW
_XF_C = 128
_XF_R = _N // _XF_C
_XFR_W = _PER_W // _XF_C
_RBLK = 32
_CH = _RBLK * _L
_CH_XFR = _CH // _XF_C
_NCHUNK = _ROWS_W // _RBLK

_mesh = plsc.VectorSubcoreMesh(core_axis_name="c", subcore_axis_name="s")


@functools.partial(
    pl.kernel,
    mesh=_mesh,
    out_type=jax.ShapeDtypeStruct((_XF_R, _XF_C), jnp.int32),
    scratch_types=[
        pltpu.VMEM((_ROWS_W, _L), jnp.int32),
        pltpu.VMEM((_XFR_W, _XF_C), jnp.int32),
    ],
    compiler_params=pltpu.CompilerParams(needs_layout_passes=False),
)
def _flatten_sc(x_hbm, xf_hbm, xv, fv):
    wid = lax.axis_index("s") * _NC + lax.axis_index("c")
    r0 = wid * _ROWS_W
    pltpu.sync_copy(x_hbm.at[pl.ds(r0, _ROWS_W), :], xv)

    def body(r, carry):
        i16 = lax.iota(jnp.int32, 16)
        p1 = r * _L + i16
        v1 = xv[r, pl.ds(0, 16)]
        plsc.store_scatter(fv, [p1 // _XF_C, p1 % _XF_C], v1)
        p2 = p1 + 4
        v2 = xv[r, pl.ds(4, 16)]
        plsc.store_scatter(fv, [p2 // _XF_C, p2 % _XF_C], v2, mask=i16 >= 12)
        return carry

    lax.fori_loop(0, _ROWS_W, body, 0)
    pltpu.sync_copy(fv, xf_hbm.at[pl.ds(wid * _XFR_W, _XFR_W), :])


@functools.partial(
    pl.kernel,
    mesh=_mesh,
    out_type=jax.ShapeDtypeStruct((_B, _L, _EMBED), jnp.float32),
    scratch_types=[
        pltpu.VMEM((_CH_XFR, _XF_C), jnp.int32),
        pltpu.VMEM((_CH, _EMBED), jnp.float32),
        pltpu.SemaphoreType.DMA,
        pltpu.SemaphoreType.DMA,
    ],
    compiler_params=pltpu.CompilerParams(use_tc_tiling_on_sc=False),
)
def _embed_sc(xf_hbm, table_hbm, out_hbm, idx_v, rows_v, gsem, ssem):
    wid = lax.axis_index("s") * _NC + lax.axis_index("c")
    xfr0 = wid * _XFR_W
    row0 = wid * _ROWS_W

    def body(i, carry):
        pltpu.sync_copy(xf_hbm.at[pl.ds(xfr0 + i * _CH_XFR, _CH_XFR), :], idx_v)
        gds = [
            pltpu.async_copy(
                table_hbm.at[idx_v.at[k]],
                rows_v.at[pl.ds(k * _XF_C, _XF_C), :],
                gsem,
            )
            for k in range(_CH_XFR)
        ]
        for d in gds:
            d.wait()
        r = row0 + i * _RBLK
        sds = [
            pltpu.async_copy(
                rows_v.at[pl.ds(k * _L, _L), :], out_hbm.at[r + k], ssem
            )
            for k in range(_RBLK)
        ]
        for d in sds:
            d.wait()
        return carry

    lax.fori_loop(0, _NCHUNK, body, 0)


def kernel(x, W):
    xf = _flatten_sc(x.astype(jnp.int32))
    return _embed_sc(xf, W)
